# baseline (speedup 1.0000x reference)
import functools

import jax
import jax.numpy as jnp
from jax import lax
from jax.experimental import pallas as pl
from jax.experimental.pallas import tpu as pltpu
from jax.experimental.pallas import tpu_sc as plsc

N_NODES = 10000
D_FEAT = 256
H1 = 128
H2 = 64
N_EDGES = 320000

NW = 32

BM = 1000
DEC_BM = 200


def _mm1_body(x_ref, w_ref, o_ref):
    o_ref[...] = jnp.dot(x_ref[...], w_ref[...],
                         preferred_element_type=jnp.float32)


def _mid_body(h_ref, nz_ref, w_ref, o_ref):
    noisy = jnp.maximum(h_ref[...], 0.0) + nz_ref[...]
    o_ref[...] = jnp.dot(noisy, w_ref[...],
                         preferred_element_type=jnp.float32)


def _dec_body(a_ref, b_ref, o_ref):
    o_ref[...] = jax.lax.dot_general(
        a_ref[...], b_ref[...],
        (((1,), (1,)), ((), ())),
        preferred_element_type=jnp.float32)


def _mm1(x, w):
    return pl.pallas_call(
        _mm1_body,
        grid=(N_NODES // BM,),
        in_specs=[
            pl.BlockSpec((BM, D_FEAT), lambda i: (i, 0)),
            pl.BlockSpec((D_FEAT, H1), lambda i: (0, 0)),
        ],
        out_specs=pl.BlockSpec((BM, H1), lambda i: (i, 0)),
        out_shape=jax.ShapeDtypeStruct((N_NODES, H1), jnp.float32),
    )(x, w)


def _mid(h1, noise, w2):
    return pl.pallas_call(
        _mid_body,
        grid=(N_NODES // BM,),
        in_specs=[
            pl.BlockSpec((BM, H1), lambda i: (i, 0)),
            pl.BlockSpec((BM, H1), lambda i: (i, 0)),
            pl.BlockSpec((H1, H2), lambda i: (0, 0)),
        ],
        out_specs=pl.BlockSpec((BM, H2), lambda i: (i, 0)),
        out_shape=jax.ShapeDtypeStruct((N_NODES, H2), jnp.float32),
    )(h1, noise, w2)


def _decoder(z):
    out = pl.pallas_call(
        _dec_body,
        grid=(N_NODES // DEC_BM,),
        in_specs=[
            pl.BlockSpec((DEC_BM, H2), lambda i: (i, 0)),
            pl.BlockSpec((N_NODES, H2), lambda i: (0, 0)),
        ],
        out_specs=pl.BlockSpec((DEC_BM, N_NODES), lambda i: (i, 0)),
        out_shape=jax.ShapeDtypeStruct((N_NODES, N_NODES), jnp.float32),
    )(z, z)
    return out.reshape(-1)


def _permute(v, idx):
    return lax.gather(
        v, idx[:, None],
        lax.GatherDimensionNumbers(
            offset_dims=(), collapsed_slice_dims=(0,),
            start_index_map=(0,)),
        (1,),
        mode=lax.GatherScatterMode.PROMISE_IN_BOUNDS)


def _make_spmm(H):
    CPT = H // NW
    EPV = 16 // CPT
    CHUNK = 8000
    NCHUNKS = N_EDGES // CHUNK

    mesh = plsc.VectorSubcoreMesh(core_axis_name="c", subcore_axis_name="s",
                                  num_cores=2, num_subcores=16)

    @functools.partial(
        pl.kernel,
        out_type=jax.ShapeDtypeStruct((NW, N_NODES * CPT), jnp.float32),
        mesh=mesh,
        scratch_types=[
            pltpu.VMEM((2 * CHUNK,), jnp.int32),
            pltpu.VMEM((2 * CHUNK,), jnp.int32),
            pltpu.VMEM((N_NODES * CPT,), jnp.float32),
            pltpu.VMEM((N_NODES * CPT,), jnp.float32),
            pltpu.SemaphoreType.DMA,
            pltpu.SemaphoreType.DMA,
        ],
        compiler_params=pltpu.CompilerParams(needs_layout_passes=False),
    )
    def spmm(src_hbm, dst_hbm, table_hbm, out_hbm,
             srcb, dstb, tbl, acc, sem_in, sem_t):
        wid = lax.axis_index("s") * 2 + lax.axis_index("c")

        iota16 = lax.iota(jnp.int32, 16)
        lmod = jnp.remainder(iota16, CPT)
        pats = [iota16 // CPT + k * EPV for k in range(CPT)]

        tload = pltpu.async_copy(table_hbm.at[wid], tbl, sem_t)

        zf = jnp.zeros((16,), jnp.float32)

        @plsc.parallel_loop(0, N_NODES * CPT // 16, unroll=8)
        def zero_acc(i):
            acc[pl.ds(i * 16, 16)] = zf

        def issue_in(c, slot):
            pltpu.async_copy(src_hbm.at[pl.ds(c * CHUNK, CHUNK)],
                             srcb.at[pl.ds(slot * CHUNK, CHUNK)], sem_in)
            pltpu.async_copy(dst_hbm.at[pl.ds(c * CHUNK, CHUNK)],
                             dstb.at[pl.ds(slot * CHUNK, CHUNK)], sem_in)

        def wait_in(slot):
            pltpu.make_async_copy(
                src_hbm.at[pl.ds(0, CHUNK)],
                srcb.at[pl.ds(slot * CHUNK, CHUNK)], sem_in).wait()
            pltpu.make_async_copy(
                dst_hbm.at[pl.ds(0, CHUNK)],
                dstb.at[pl.ds(slot * CHUNK, CHUNK)], sem_in).wait()

        issue_in(0, 0)
        tload.wait()

        def chunk_body(c, _):
            slot = lax.rem(c, 2)
            nxt = lax.rem(c + 1, NCHUNKS)
            wait_in(slot)
            issue_in(nxt, lax.rem(c + 1, 2))

            off0 = slot * CHUNK

            @plsc.parallel_loop(0, CHUNK // 16, unroll=4)
            def edge_vec(i):
                s = srcb[pl.ds(off0 + i * 16, 16)]
                d = dstb[pl.ds(off0 + i * 16, 16)]
                sb = s * CPT
                db = d * CPT
                for k in range(CPT):
                    v = plsc.load_gather(tbl, [sb + k])
                    plsc.addupdate_scatter(acc, [db + k], v)
            return _

        lax.fori_loop(0, NCHUNKS, chunk_body, None)

        wait_in(lax.rem(jnp.int32(NCHUNKS), 2))

        pltpu.sync_copy(acc, out_hbm.at[wid])

    def wrapped(src, dst, table):
        table_p = table.reshape(N_NODES, NW, CPT).transpose(1, 0, 2)
        out_p = spmm(src, dst, table_p.reshape(NW, N_NODES * CPT))
        return (out_p.reshape(NW, N_NODES, CPT)
                .transpose(1, 0, 2).reshape(N_NODES, H))

    return wrapped


_spmm128 = _make_spmm(H1)
_spmm64 = _make_spmm(H2)


def kernel(features, edge_index, W1, W2):
    src = edge_index[0]
    dst = edge_index[1]
    xw = _mm1(features, W1)
    h1 = _spmm128(src, dst, xw)
    noise = 0.1 * jax.random.normal(jax.random.key(42), (N_NODES, H1),
                                    dtype=jnp.float32)
    hw = _mid(h1, noise, W2)
    z = _spmm64(src, dst, hw)
    return _decoder(z)


# speedup vs baseline: 1.0125x; 1.0125x over previous
import functools

import jax
import jax.numpy as jnp
from jax import lax
from jax.experimental import pallas as pl
from jax.experimental.pallas import tpu as pltpu
from jax.experimental.pallas import tpu_sc as plsc

N_NODES = 10000
D_FEAT = 256
H1 = 128
H2 = 64
N_EDGES = 320000

NW = 32

BM = 1000
DEC_BM = 200


def _mm1_body(x_ref, w_ref, o_ref):
    o_ref[...] = jnp.dot(x_ref[...], w_ref[...],
                         preferred_element_type=jnp.float32)


def _mid_body(h_ref, nz_ref, w_ref, o_ref):
    noisy = jnp.maximum(h_ref[...], 0.0) + nz_ref[...]
    o_ref[...] = jnp.dot(noisy, w_ref[...],
                         preferred_element_type=jnp.float32)


def _dec_body(a_ref, b_ref, o_ref):
    o_ref[...] = jax.lax.dot_general(
        a_ref[...], b_ref[...],
        (((1,), (1,)), ((), ())),
        preferred_element_type=jnp.float32)


def _mm1(x, w):
    return pl.pallas_call(
        _mm1_body,
        grid=(N_NODES // BM,),
        in_specs=[
            pl.BlockSpec((BM, D_FEAT), lambda i: (i, 0)),
            pl.BlockSpec((D_FEAT, H1), lambda i: (0, 0)),
        ],
        out_specs=pl.BlockSpec((BM, H1), lambda i: (i, 0)),
        out_shape=jax.ShapeDtypeStruct((N_NODES, H1), jnp.float32),
    )(x, w)


def _mid(h1, noise, w2):
    return pl.pallas_call(
        _mid_body,
        grid=(N_NODES // BM,),
        in_specs=[
            pl.BlockSpec((BM, H1), lambda i: (i, 0)),
            pl.BlockSpec((BM, H1), lambda i: (i, 0)),
            pl.BlockSpec((H1, H2), lambda i: (0, 0)),
        ],
        out_specs=pl.BlockSpec((BM, H2), lambda i: (i, 0)),
        out_shape=jax.ShapeDtypeStruct((N_NODES, H2), jnp.float32),
    )(h1, noise, w2)


def _decoder(z):
    out = pl.pallas_call(
        _dec_body,
        grid=(N_NODES // DEC_BM,),
        in_specs=[
            pl.BlockSpec((DEC_BM, H2), lambda i: (i, 0)),
            pl.BlockSpec((N_NODES, H2), lambda i: (0, 0)),
        ],
        out_specs=pl.BlockSpec((DEC_BM, N_NODES), lambda i: (i, 0)),
        out_shape=jax.ShapeDtypeStruct((N_NODES, N_NODES), jnp.float32),
    )(z, z)
    return out.reshape(-1)


def _permute(v, idx):
    return lax.gather(
        v, idx[:, None],
        lax.GatherDimensionNumbers(
            offset_dims=(), collapsed_slice_dims=(0,),
            start_index_map=(0,)),
        (1,),
        mode=lax.GatherScatterMode.PROMISE_IN_BOUNDS)


def _make_spmm(H):
    CPT = H // NW
    EPV = 16 // CPT
    CHUNK = 8000
    NCHUNKS = N_EDGES // CHUNK

    mesh = plsc.VectorSubcoreMesh(core_axis_name="c", subcore_axis_name="s",
                                  num_cores=2, num_subcores=16)

    @functools.partial(
        pl.kernel,
        out_type=jax.ShapeDtypeStruct((NW, N_NODES * CPT), jnp.float32),
        mesh=mesh,
        scratch_types=[
            pltpu.VMEM((2 * CHUNK,), jnp.int32),
            pltpu.VMEM((2 * CHUNK,), jnp.int32),
            pltpu.VMEM((N_NODES * CPT,), jnp.float32),
            pltpu.VMEM((N_NODES * CPT,), jnp.float32),
            pltpu.SemaphoreType.DMA,
            pltpu.SemaphoreType.DMA,
        ],
        compiler_params=pltpu.CompilerParams(needs_layout_passes=False),
    )
    def spmm(src_hbm, dst_hbm, table_hbm, out_hbm,
             srcb, dstb, tbl, acc, sem_in, sem_t):
        wid = lax.axis_index("s") * 2 + lax.axis_index("c")

        iota16 = lax.iota(jnp.int32, 16)
        lmod = jnp.remainder(iota16, CPT)
        pats = [iota16 // CPT + k * EPV for k in range(CPT)]

        tload = pltpu.async_copy(table_hbm.at[wid], tbl, sem_t)

        zf = jnp.zeros((16,), jnp.float32)

        @plsc.parallel_loop(0, N_NODES * CPT // 16, unroll=8)
        def zero_acc(i):
            acc[pl.ds(i * 16, 16)] = zf

        def issue_in(c, slot):
            pltpu.async_copy(src_hbm.at[pl.ds(c * CHUNK, CHUNK)],
                             srcb.at[pl.ds(slot * CHUNK, CHUNK)], sem_in)
            pltpu.async_copy(dst_hbm.at[pl.ds(c * CHUNK, CHUNK)],
                             dstb.at[pl.ds(slot * CHUNK, CHUNK)], sem_in)

        def wait_in(slot):
            pltpu.make_async_copy(
                src_hbm.at[pl.ds(0, CHUNK)],
                srcb.at[pl.ds(slot * CHUNK, CHUNK)], sem_in).wait()
            pltpu.make_async_copy(
                dst_hbm.at[pl.ds(0, CHUNK)],
                dstb.at[pl.ds(slot * CHUNK, CHUNK)], sem_in).wait()

        issue_in(0, 0)
        tload.wait()

        def chunk_body(c, _):
            slot = lax.rem(c, 2)
            nxt = lax.rem(c + 1, NCHUNKS)
            wait_in(slot)
            issue_in(nxt, lax.rem(c + 1, 2))

            off0 = slot * CHUNK

            @plsc.parallel_loop(0, CHUNK // 16, unroll=2)
            def edge_vec(i):
                s = srcb[pl.ds(off0 + i * 16, 16)]
                d = dstb[pl.ds(off0 + i * 16, 16)]
                sb = s * CPT
                db = d * CPT
                for k in range(CPT):
                    v = plsc.load_gather(tbl, [sb + k])
                    plsc.addupdate_scatter(acc, [db + k], v)
            return _

        lax.fori_loop(0, NCHUNKS, chunk_body, None)

        wait_in(lax.rem(jnp.int32(NCHUNKS), 2))

        pltpu.sync_copy(acc, out_hbm.at[wid])

    def wrapped(src, dst, table):
        table_p = table.reshape(N_NODES, NW, CPT).transpose(1, 0, 2)
        out_p = spmm(src, dst, table_p.reshape(NW, N_NODES * CPT))
        return (out_p.reshape(NW, N_NODES, CPT)
                .transpose(1, 0, 2).reshape(N_NODES, H))

    return wrapped


_spmm128 = _make_spmm(H1)
_spmm64 = _make_spmm(H2)


def kernel(features, edge_index, W1, W2):
    src = edge_index[0]
    dst = edge_index[1]
    xw = _mm1(features, W1)
    h1 = _spmm128(src, dst, xw)
    noise = 0.1 * jax.random.normal(jax.random.key(42), (N_NODES, H1),
                                    dtype=jnp.float32)
    hw = _mid(h1, noise, W2)
    z = _spmm64(src, dst, hw)
    return _decoder(z)


# speedup vs baseline: 1.2122x; 1.1972x over previous
import functools

import jax
import jax.numpy as jnp
from jax import lax
from jax.experimental import pallas as pl
from jax.experimental.pallas import tpu as pltpu
from jax.experimental.pallas import tpu_sc as plsc

N_NODES = 10000
D_FEAT = 256
H1 = 128
H2 = 64
N_EDGES = 320000

NW = 32

BM = 1000
DEC_BM = 200


def _mm1_body(x_ref, w_ref, o_ref):
    o_ref[...] = jnp.dot(x_ref[...], w_ref[...],
                         preferred_element_type=jnp.float32)


def _mid_body(h_ref, nz_ref, w_ref, o_ref):
    noisy = jnp.maximum(h_ref[...], 0.0) + nz_ref[...]
    o_ref[...] = jnp.dot(noisy, w_ref[...],
                         preferred_element_type=jnp.float32)


def _dec_body(a_ref, b_ref, o_ref):
    o_ref[...] = jax.lax.dot_general(
        a_ref[...], b_ref[...],
        (((1,), (1,)), ((), ())),
        preferred_element_type=jnp.float32)


def _mm1(x, w):
    return pl.pallas_call(
        _mm1_body,
        grid=(N_NODES // BM,),
        in_specs=[
            pl.BlockSpec((BM, D_FEAT), lambda i: (i, 0)),
            pl.BlockSpec((D_FEAT, H1), lambda i: (0, 0)),
        ],
        out_specs=pl.BlockSpec((BM, H1), lambda i: (i, 0)),
        out_shape=jax.ShapeDtypeStruct((N_NODES, H1), jnp.float32),
    )(x, w)


def _mid(h1, noise, w2):
    return pl.pallas_call(
        _mid_body,
        grid=(N_NODES // BM,),
        in_specs=[
            pl.BlockSpec((BM, H1), lambda i: (i, 0)),
            pl.BlockSpec((BM, H1), lambda i: (i, 0)),
            pl.BlockSpec((H1, H2), lambda i: (0, 0)),
        ],
        out_specs=pl.BlockSpec((BM, H2), lambda i: (i, 0)),
        out_shape=jax.ShapeDtypeStruct((N_NODES, H2), jnp.float32),
    )(h1, noise, w2)


def _decoder(z):
    out = pl.pallas_call(
        _dec_body,
        grid=(N_NODES // DEC_BM,),
        in_specs=[
            pl.BlockSpec((DEC_BM, H2), lambda i: (i, 0)),
            pl.BlockSpec((N_NODES, H2), lambda i: (0, 0)),
        ],
        out_specs=pl.BlockSpec((DEC_BM, N_NODES), lambda i: (i, 0)),
        out_shape=jax.ShapeDtypeStruct((N_NODES, N_NODES), jnp.float32),
    )(z, z)
    return out.reshape(-1)


def _permute(v, idx):
    return lax.gather(
        v, idx[:, None],
        lax.GatherDimensionNumbers(
            offset_dims=(), collapsed_slice_dims=(0,),
            start_index_map=(0,)),
        (1,),
        mode=lax.GatherScatterMode.PROMISE_IN_BOUNDS)


def _make_spmm(H):
    CPT = H // NW
    EPV = 16 // CPT
    CHUNK = 16000
    NCHUNKS = N_EDGES // CHUNK

    mesh = plsc.VectorSubcoreMesh(core_axis_name="c", subcore_axis_name="s",
                                  num_cores=2, num_subcores=16)

    @functools.partial(
        pl.kernel,
        out_type=jax.ShapeDtypeStruct((NW, N_NODES * CPT), jnp.float32),
        mesh=mesh,
        scratch_types=[
            pltpu.VMEM((2 * CHUNK,), jnp.uint32),
            pltpu.VMEM((N_NODES * CPT,), jnp.float32),
            pltpu.VMEM((N_NODES * CPT,), jnp.float32),
            pltpu.SemaphoreType.DMA,
            pltpu.SemaphoreType.DMA,
        ],
        compiler_params=pltpu.CompilerParams(needs_layout_passes=False),
    )
    def spmm(ep_hbm, table_hbm, out_hbm,
             epb, tbl, acc, sem_in, sem_t):
        wid = lax.axis_index("s") * 2 + lax.axis_index("c")

        iota16 = lax.iota(jnp.int32, 16)
        lmod = jnp.remainder(iota16, CPT).astype(jnp.uint32)
        pats = [iota16 // CPT + k * EPV for k in range(CPT)]

        tload = pltpu.async_copy(table_hbm.at[wid], tbl, sem_t)

        zf = jnp.zeros((16,), jnp.float32)

        @plsc.parallel_loop(0, N_NODES * CPT // 16, unroll=8)
        def zero_acc(i):
            acc[pl.ds(i * 16, 16)] = zf

        def issue_in(c, slot):
            pltpu.async_copy(ep_hbm.at[pl.ds(c * CHUNK, CHUNK)],
                             epb.at[pl.ds(slot * CHUNK, CHUNK)], sem_in)

        def wait_in(slot):
            pltpu.make_async_copy(
                ep_hbm.at[pl.ds(0, CHUNK)],
                epb.at[pl.ds(slot * CHUNK, CHUNK)], sem_in).wait()

        issue_in(0, 0)
        tload.wait()

        def chunk_body(c, _):
            slot = lax.rem(c, 2)
            nxt = lax.rem(c + 1, NCHUNKS)
            wait_in(slot)
            issue_in(nxt, lax.rem(c + 1, 2))

            off0 = slot * CHUNK

            @plsc.parallel_loop(0, CHUNK // 16, unroll=2)
            def edge_vec(i):
                p = epb[pl.ds(off0 + i * 16, 16)]
                for k in range(CPT):
                    pp = _permute(p, pats[k])
                    si = plsc.bitcast((pp >> 16) + lmod, jnp.int32)
                    di = plsc.bitcast((pp & 0xFFFF) + lmod, jnp.int32)
                    v = plsc.load_gather(tbl, [si])
                    plsc.addupdate_scatter(acc, [di], v)
            return _

        lax.fori_loop(0, NCHUNKS, chunk_body, None)

        wait_in(lax.rem(jnp.int32(NCHUNKS), 2))

        pltpu.sync_copy(acc, out_hbm.at[wid])

    def wrapped(ep, table):
        table_p = table.reshape(N_NODES, NW, CPT).transpose(1, 0, 2)
        out_p = spmm(ep, table_p.reshape(NW, N_NODES * CPT))
        return (out_p.reshape(NW, N_NODES, CPT)
                .transpose(1, 0, 2).reshape(N_NODES, H))

    def pack(src, dst):
        s = src.astype(jnp.uint32) * CPT
        d = dst.astype(jnp.uint32) * CPT
        return (s << 16) | d

    wrapped.pack = pack

    return wrapped


_spmm128 = _make_spmm(H1)
_spmm64 = _make_spmm(H2)


def kernel(features, edge_index, W1, W2):
    src = edge_index[0]
    dst = edge_index[1]
    ep128 = _spmm128.pack(src, dst)
    ep64 = _spmm64.pack(src, dst)
    xw = _mm1(features, W1)
    h1 = _spmm128(ep128, xw)
    noise = 0.1 * jax.random.normal(jax.random.key(42), (N_NODES, H1),
                                    dtype=jnp.float32)
    hw = _mid(h1, noise, W2)
    z = _spmm64(ep64, hw)
    return _decoder(z)
